# Initial kernel scaffold; baseline (speedup 1.0000x reference)
import functools

import jax
import jax.numpy as jnp
from jax import lax
from jax.experimental import pallas as pl
from jax.experimental.pallas import tpu as pltpu
from jax.experimental.pallas import tpu_sc as plsc

_B, _N, _C = 64, 8192, 16
_L = 16
_V = _N // _L
_NBINS = 256
_NPASS = 4
_NW = 32
_ROWS_PER_W = _B // _NW
_CHUNK = 128
_SUPER = 1024
_MIN32 = jnp.int32(-2147483648)


def _body(keys_hbm, xr_hbm, out_hbm, keyA, keyB, payA, payB, off, idxbuf,
          stag0, stag1, sem0, sem1):
    iota = lax.iota(jnp.int32, _L)
    ones = jnp.ones((_L,), jnp.int32)
    zeros = jnp.zeros((_L,), jnp.int32)
    wid = lax.axis_index("s") * 2 + lax.axis_index("c")

    def do_row(r, _):
        row = wid * _ROWS_PER_W + r
        rowbase = row * _N
        pltpu.sync_copy(keys_hbm.at[pl.ds(rowbase, _N)], keyB)

        def init_body(v, _):
            k = keyB[pl.ds(v * _L, _L)]
            k = jnp.where(k == _MIN32, 0, k)
            t = k ^ _MIN32
            d = jnp.where(k >= 0, ~t, k)
            addr = (v & 31) * 256 + (v >> 5) + iota * _L
            plsc.store_scatter(keyA, [addr], d)
            payA[pl.ds(v * _L, _L)] = iota * _V + v
        lax.fori_loop(0, _V, init_body, None, unroll=2)

        for p in range(_NPASS):
            shift = jnp.full((_L,), 8 * p, jnp.int32)
            ik, ip, ok_, op_ = ((keyA, payA, keyB, payB) if p % 2 == 0
                                else (keyB, payB, keyA, payA))

            def zero_body(b, _):
                off[pl.ds(b * _L, _L)] = zeros
            lax.fori_loop(0, _NBINS, zero_body, None, unroll=4)

            def hist_body(v, _):
                d = ik[pl.ds(v * _L, _L)]
                dig = lax.shift_right_logical(d, shift) & 255
                plsc.addupdate_scatter(off, [dig * _L + iota], ones)
            lax.fori_loop(0, _V, hist_body, None, unroll=4)

            def scan_body(b, carry):
                h = off[pl.ds(b * _L, _L)]
                cs = plsc.cumsum(h)
                off[pl.ds(b * _L, _L)] = cs - h + carry
                return carry + jnp.sum(h)
            lax.fori_loop(0, _NBINS, scan_body, jnp.int32(0), unroll=2)

            def perm_body(v, _):
                d = ik[pl.ds(v * _L, _L)]
                pv = ip[pl.ds(v * _L, _L)]
                dig = lax.shift_right_logical(d, shift) & 255
                addr = dig * _L + iota
                s = plsc.load_gather(off, [addr])
                plsc.addupdate_scatter(off, [addr], ones)
                a = ((s & 511) << 4) | (s >> 9)
                if p < _NPASS - 1:
                    plsc.store_scatter(ok_, [a], d)
                plsc.store_scatter(op_, [a], pv)
            lax.fori_loop(0, _V, perm_body, None, unroll=2)

        def untr_body(w, _):
            base = (w & 31) * 256 + (w >> 5)
            g = plsc.load_gather(payA, [base + iota * _L])
            idxbuf[w >> 3, pl.ds((w & 7) * _L, _L)] = g + rowbase
        lax.fori_loop(0, _V, untr_body, None, unroll=2)

        def issue(sc, stag, sem):
            def is_body(j, _):
                pltpu.async_copy(
                    xr_hbm.at[idxbuf.at[sc * 8 + j]],
                    stag.at[pl.ds(j * _CHUNK, _CHUNK)], sem)
            lax.fori_loop(0, 8, is_body, None)

        def drain(stag, sem):
            pltpu.make_async_copy(out_hbm.at[pl.ds(0, _SUPER)], stag, sem).wait()

        def store(sc, stag):
            pltpu.sync_copy(stag, out_hbm.at[pl.ds(rowbase + sc * _SUPER, _SUPER)])

        issue(0, stag0, sem0)

        def g_body(cc, _):
            sc0 = cc * 2
            issue(sc0 + 1, stag1, sem1)
            drain(stag0, sem0)
            store(sc0, stag0)

            @pl.when(cc < (_N // _SUPER) // 2 - 1)
            def _():
                issue(sc0 + 2, stag0, sem0)
            drain(stag1, sem1)
            store(sc0 + 1, stag1)
        lax.fori_loop(0, (_N // _SUPER) // 2, g_body, None)

    lax.fori_loop(0, _ROWS_PER_W, do_row, None)


_sc_sort = functools.partial(
    pl.kernel,
    out_type=jax.ShapeDtypeStruct((_B * _N, _C), jnp.float32),
    mesh=plsc.VectorSubcoreMesh(core_axis_name="c", subcore_axis_name="s",
                                num_cores=2, num_subcores=16),
    scratch_types=[
        pltpu.VMEM((_N,), jnp.int32),
        pltpu.VMEM((_N,), jnp.int32),
        pltpu.VMEM((_N,), jnp.int32),
        pltpu.VMEM((_N,), jnp.int32),
        pltpu.VMEM((_NBINS * _L,), jnp.int32),
        pltpu.VMEM((_N // _CHUNK, _CHUNK), jnp.int32),
        pltpu.VMEM((_SUPER, _C), jnp.float32),
        pltpu.VMEM((_SUPER, _C), jnp.float32),
        pltpu.SemaphoreType.DMA,
        pltpu.SemaphoreType.DMA,
    ],
)(_body)


@jax.jit
def kernel(x):
    keys = lax.bitcast_convert_type(x[:, :, 0], jnp.int32).reshape(-1)
    xr = x.reshape(_B * _N, _C)
    out = _sc_sort(keys, xr)
    return out.reshape(_B, _N, _C)


# speedup vs baseline: 1.0895x; 1.0895x over previous
import functools

import jax
import jax.numpy as jnp
from jax import lax
from jax.experimental import pallas as pl
from jax.experimental.pallas import tpu as pltpu
from jax.experimental.pallas import tpu_sc as plsc

_B, _N, _C = 64, 8192, 16
_L = 16
_V = _N // _L
_NBINS = 256
_NPASS = 4
_NW = 32
_ROWS_PER_W = _B // _NW
_CHUNK = 128
_SUPER = 1024
_MIN32 = -2147483648


def _body(keys_hbm, xr_hbm, out_hbm, keyA, keyB, payA, payB, off, idxbuf,
          stag0, stag1, sem0, sem1):
    iota = lax.iota(jnp.int32, _L)
    ones = jnp.ones((_L,), jnp.int32)
    zeros = jnp.zeros((_L,), jnp.int32)
    wid = lax.axis_index("s") * 2 + lax.axis_index("c")

    def do_row(r, _):
        row = wid * _ROWS_PER_W + r
        rowbase = row * _N
        pltpu.sync_copy(keys_hbm.at[pl.ds(rowbase, _N)], keyB)

        def init_body(v, _):
            k = keyB[pl.ds(v * _L, _L)]
            k = jnp.where(k == _MIN32, 0, k)
            t = k ^ _MIN32
            d = jnp.where(k >= 0, ~t, k)
            addr = (v & 31) * 256 + (v >> 5) + iota * _L
            plsc.store_scatter(keyA, [addr], d)
            payA[pl.ds(v * _L, _L)] = iota * _V + v
        lax.fori_loop(0, _V, init_body, None, unroll=2)

        for p in range(_NPASS):
            shift = jnp.full((_L,), 8 * p, jnp.int32)
            ik, ip, ok_, op_ = ((keyA, payA, keyB, payB) if p % 2 == 0
                                else (keyB, payB, keyA, payA))

            def zero_body(b, _):
                off[pl.ds(b * _L, _L)] = zeros
            lax.fori_loop(0, _NBINS, zero_body, None, unroll=4)

            def hist_body(v, _):
                d = ik[pl.ds(v * _L, _L)]
                dig = lax.shift_right_logical(d, shift) & 255
                plsc.addupdate_scatter(off, [dig * _L + iota], ones)
            lax.fori_loop(0, _V, hist_body, None, unroll=4)

            def scan_body(b, carry):
                h = off[pl.ds(b * _L, _L)]
                cs = plsc.cumsum(h)
                off[pl.ds(b * _L, _L)] = cs - h + carry
                return carry + jnp.sum(h)
            lax.fori_loop(0, _NBINS, scan_body, jnp.int32(0), unroll=2)

            def perm_body(v, _):
                d = ik[pl.ds(v * _L, _L)]
                pv = ip[pl.ds(v * _L, _L)]
                dig = lax.shift_right_logical(d, shift) & 255
                addr = dig * _L + iota
                s = plsc.load_gather(off, [addr])
                plsc.addupdate_scatter(off, [addr], ones)
                a = ((s & 511) << 4) | (s >> 9)
                if p < _NPASS - 1:
                    plsc.store_scatter(ok_, [a], d)
                plsc.store_scatter(op_, [a], pv)
            lax.fori_loop(0, _V, perm_body, None, unroll=2)

        def untr_body(w, _):
            base = (w & 31) * 256 + (w >> 5)
            g = plsc.load_gather(payA, [base + iota * _L])
            idxbuf[w >> 3, pl.ds((w & 7) * _L, _L)] = g + rowbase
        lax.fori_loop(0, _V, untr_body, None, unroll=2)

        def issue(sc, stag, sem):
            def is_body(j, _):
                pltpu.async_copy(
                    xr_hbm.at[idxbuf.at[sc * 8 + j]],
                    stag.at[pl.ds(j * _CHUNK, _CHUNK)], sem)
            lax.fori_loop(0, 8, is_body, None)

        def drain(stag, sem):
            pltpu.make_async_copy(out_hbm.at[pl.ds(0, _SUPER)], stag, sem).wait()

        def store(sc, stag):
            pltpu.sync_copy(stag, out_hbm.at[pl.ds(rowbase + sc * _SUPER, _SUPER)])

        issue(0, stag0, sem0)

        def g_body(cc, _):
            sc0 = cc * 2
            issue(sc0 + 1, stag1, sem1)
            drain(stag0, sem0)
            store(sc0, stag0)

            @pl.when(cc < (_N // _SUPER) // 2 - 1)
            def _():
                issue(sc0 + 2, stag0, sem0)
            drain(stag1, sem1)
            store(sc0 + 1, stag1)
        lax.fori_loop(0, (_N // _SUPER) // 2, g_body, None)

    lax.fori_loop(0, _ROWS_PER_W, do_row, None)


_sc_sort = functools.partial(
    pl.kernel,
    out_type=jax.ShapeDtypeStruct((_B * _N, _C), jnp.float32),
    mesh=plsc.VectorSubcoreMesh(core_axis_name="c", subcore_axis_name="s",
                                num_cores=2, num_subcores=16),
    scratch_types=[
        pltpu.VMEM((_N,), jnp.int32),
        pltpu.VMEM((_N,), jnp.int32),
        pltpu.VMEM((_N,), jnp.int32),
        pltpu.VMEM((_N,), jnp.int32),
        pltpu.VMEM((_NBINS * _L,), jnp.int32),
        pltpu.VMEM((_N // _CHUNK, _CHUNK), jnp.int32),
        pltpu.VMEM((_SUPER, _C), jnp.float32),
        pltpu.VMEM((_SUPER, _C), jnp.float32),
        pltpu.SemaphoreType.DMA,
        pltpu.SemaphoreType.DMA,
    ],
    compiler_params=pltpu.CompilerParams(needs_layout_passes=False,
                                         use_tc_tiling_on_sc=False),
)(_body)


@jax.jit
def kernel(x):
    keys = lax.bitcast_convert_type(x[:, :, 0], jnp.int32).reshape(-1)
    xr = x.reshape(_B * _N, _C)
    out = _sc_sort(keys, xr)
    return out.reshape(_B, _N, _C)


# speedup vs baseline: 2.5504x; 2.3409x over previous
import functools

import jax
import jax.numpy as jnp
from jax import lax
from jax.experimental import pallas as pl
from jax.experimental.pallas import tpu as pltpu
from jax.experimental.pallas import tpu_sc as plsc

_B, _N, _C = 64, 8192, 16
_L = 16
_V = _N // _L
_NBINS = 256
_NPASS = 4
_NW = 32
_ROWS_PER_W = _B // _NW
_HALF = _N * 8
_CHW = 8192
_MIN32 = -2147483648


def _body(x5_hbm, out5_hbm, inblk, keyA, keyB, payA, payB, off, pidx,
          ob0, ob1, sem0, sem1):
    iota = lax.iota(jnp.int32, _L)
    ones = jnp.ones((_L,), jnp.int32)
    zeros = jnp.zeros((_L,), jnp.int32)
    wid = lax.axis_index("s") * 2 + lax.axis_index("c")

    def do_row(r, _):
        b = wid * _ROWS_PER_W + r

        pltpu.sync_copy(x5_hbm.at[b, 0], inblk)

        def init_body(v, _):
            kf = inblk[pl.ds((v >> 3) * 1024 + (v & 7) * _L, _L)]
            k = plsc.bitcast(kf, jnp.int32)
            k = jnp.where(k == _MIN32, 0, k)
            t = k ^ _MIN32
            d = jnp.where(k >= 0, ~t, k)
            addr = (v & 31) * 256 + (v >> 5) + iota * _L
            plsc.store_scatter(keyA, [addr], d)
            payA[pl.ds(v * _L, _L)] = iota * _V + v
        lax.fori_loop(0, _V, init_body, None, unroll=2)

        for p in range(_NPASS):
            shift = jnp.full((_L,), 8 * p, jnp.int32)
            ik, ip, ok_, op_ = ((keyA, payA, keyB, payB) if p % 2 == 0
                                else (keyB, payB, keyA, payA))

            def zero_body(bb, _):
                off[pl.ds(bb * _L, _L)] = zeros
            lax.fori_loop(0, _NBINS, zero_body, None, unroll=4)

            def hist_body(v, _):
                d = ik[pl.ds(v * _L, _L)]
                dig = lax.shift_right_logical(d, shift) & 255
                plsc.addupdate_scatter(off, [dig * _L + iota], ones)
            lax.fori_loop(0, _V, hist_body, None, unroll=4)

            def scan_body(bb, carry):
                h = off[pl.ds(bb * _L, _L)]
                cs = plsc.cumsum(h)
                off[pl.ds(bb * _L, _L)] = cs - h + carry
                return carry + jnp.sum(h)
            lax.fori_loop(0, _NBINS, scan_body, jnp.int32(0), unroll=2)

            def perm_body(v, _):
                d = ik[pl.ds(v * _L, _L)]
                pv = ip[pl.ds(v * _L, _L)]
                dig = lax.shift_right_logical(d, shift) & 255
                addr = dig * _L + iota
                s = plsc.load_gather(off, [addr])
                plsc.addupdate_scatter(off, [addr], ones)
                a = ((s & 511) << 4) | (s >> 9)
                if p < _NPASS - 1:
                    plsc.store_scatter(ok_, [a], d)
                plsc.store_scatter(op_, [a], pv)
            lax.fori_loop(0, _V, perm_body, None, unroll=2)

        def untr_body(w, _):
            base = (w & 31) * 256 + (w >> 5)
            g = plsc.load_gather(payA, [base + iota * _L])
            pidx[pl.ds(w * _L, _L)] = ((g >> 7) << 10) | (g & 127)
        lax.fori_loop(0, _V, untr_body, None, unroll=2)

        def gather_chunk(nt2, ob):
            def u_body(u, _):
                obase = u * 1024
                sbase = nt2 * 1024 + u * 128
                for vv in range(8):
                    bases = pidx[pl.ds(sbase + vv * _L, _L)]
                    for c in range(8):
                        g = plsc.load_gather(inblk, [bases + c * 128])
                        ob[pl.ds(obase + c * 128 + vv * _L, _L)] = g
            lax.fori_loop(0, 8, u_body, None)

        for ct in range(2):
            if ct == 1:
                pltpu.sync_copy(x5_hbm.at[b, 1], inblk)

            def dst(nt2):
                return out5_hbm.at[b, ct, pl.ds(nt2 * _CHW, _CHW)]

            def g_body(cc, _):
                nt2 = cc * 2

                @pl.when(cc > 0)
                def _():
                    pltpu.make_async_copy(ob1, dst(nt2 - 1), sem1).wait()
                gather_chunk(nt2, ob0)
                pltpu.async_copy(ob0, dst(nt2), sem0)
                gather_chunk(nt2 + 1, ob1)
                pltpu.make_async_copy(ob0, dst(nt2), sem0).wait()
                pltpu.async_copy(ob1, dst(nt2 + 1), sem1)
            lax.fori_loop(0, 4, g_body, None)
            pltpu.make_async_copy(ob1, dst(7), sem1).wait()

    lax.fori_loop(0, _ROWS_PER_W, do_row, None)


_sc_sort = functools.partial(
    pl.kernel,
    out_type=jax.ShapeDtypeStruct((_B, 2, _HALF), jnp.float32),
    mesh=plsc.VectorSubcoreMesh(core_axis_name="c", subcore_axis_name="s",
                                num_cores=2, num_subcores=16),
    scratch_types=[
        pltpu.VMEM((_HALF,), jnp.float32),
        pltpu.VMEM((_N,), jnp.int32),
        pltpu.VMEM((_N,), jnp.int32),
        pltpu.VMEM((_N,), jnp.int32),
        pltpu.VMEM((_N,), jnp.int32),
        pltpu.VMEM((_NBINS * _L,), jnp.int32),
        pltpu.VMEM((_N,), jnp.int32),
        pltpu.VMEM((_CHW,), jnp.float32),
        pltpu.VMEM((_CHW,), jnp.float32),
        pltpu.SemaphoreType.DMA,
        pltpu.SemaphoreType.DMA,
    ],
    compiler_params=pltpu.CompilerParams(needs_layout_passes=False,
                                         use_tc_tiling_on_sc=False),
)(_body)


@jax.jit
def kernel(x):
    xt = lax.transpose(x, (0, 2, 1))
    x5 = (xt.reshape(_B, 2, 8, _N // 128, 128)
            .transpose(0, 1, 3, 2, 4).reshape(_B, 2, _HALF))
    o5 = _sc_sort(x5)
    out = (o5.reshape(_B, 2, _N // 128, 8, 128)
             .transpose(0, 1, 3, 2, 4).reshape(_B, _C, _N))
    return lax.transpose(out, (0, 2, 1))
